# baseline (speedup 1.0000x reference)
import jax
import jax.numpy as jnp
from jax import lax
from jax.experimental import pallas as pl
from jax.experimental.pallas import tpu as pltpu
from jax.experimental.pallas import tpu_sc as plsc

BATCH = 16384
EMB = 128
NC, NS = 2, 16
NW = NC * NS
CHUNK = 128
PIPE = 2
CB = BATCH // PIPE
B_PER_W = CB // NW
NCH = B_PER_W // CHUNK


def _make_gather_body(p):
    def _gather_body(users_hbm, items_hbm, utab_hbm, mtab_hbm,
                     uout_hbm, iout_hbm, idxu_v, idxi_v, rowsu_v, rowsi_v,
                     sem_u, sem_i, sem_out):
        wid = lax.axis_index("s") * NC + lax.axis_index("c")
        src = p * CB + wid * B_PER_W
        dst = wid * B_PER_W
        iu = pltpu.async_copy(users_hbm.at[pl.ds(src, B_PER_W)], idxu_v,
                              sem_u)
        ii = pltpu.async_copy(items_hbm.at[pl.ds(src, B_PER_W)], idxi_v,
                              sem_i)
        iu.wait()
        ii.wait()
        ucopies = [pltpu.async_copy(
            utab_hbm.at[idxu_v.at[pl.ds(j * CHUNK, CHUNK)]], rowsu_v.at[j],
            sem_u) for j in range(NCH)]
        icopies = [pltpu.async_copy(
            mtab_hbm.at[idxi_v.at[pl.ds(j * CHUNK, CHUNK)]], rowsi_v.at[j],
            sem_i) for j in range(NCH)]
        outs = []
        for j in range(NCH):
            ucopies[j].wait()
            outs.append(pltpu.async_copy(
                rowsu_v.at[j], uout_hbm.at[pl.ds(dst + j * CHUNK, CHUNK)],
                sem_out))
        for j in range(NCH):
            icopies[j].wait()
            outs.append(pltpu.async_copy(
                rowsi_v.at[j], iout_hbm.at[pl.ds(dst + j * CHUNK, CHUNK)],
                sem_out))
        for c in outs:
            c.wait()
    return _gather_body


def _sc_gather(p, users, items, user_table, movie_table):
    mesh = plsc.VectorSubcoreMesh(core_axis_name="c", subcore_axis_name="s",
                                  num_cores=NC, num_subcores=NS)
    emb = jax.ShapeDtypeStruct((CB, EMB), jnp.float32)
    run = pl.kernel(
        _make_gather_body(p),
        mesh=mesh,
        out_type=[emb, emb],
        scratch_types=[
            pltpu.VMEM((B_PER_W,), jnp.int32),
            pltpu.VMEM((B_PER_W,), jnp.int32),
            pltpu.VMEM((NCH, CHUNK, EMB), jnp.float32),
            pltpu.VMEM((NCH, CHUNK, EMB), jnp.float32),
            pltpu.SemaphoreType.DMA,
            pltpu.SemaphoreType.DMA,
            pltpu.SemaphoreType.DMA,
        ],
    )
    return run(users, items, user_table, movie_table)


def _mlp_body(u_ref, i_ref, w1a_ref, w1b_ref, b1_ref, w2_ref, b2_ref,
              wout_ref, bout_ref, out_ref):
    h = jnp.dot(u_ref[:], w1a_ref[:], preferred_element_type=jnp.float32)
    h = h + jnp.dot(i_ref[:], w1b_ref[:], preferred_element_type=jnp.float32)
    h = jnp.maximum(h + b1_ref[:], 0.0)
    h = jnp.maximum(
        jnp.dot(h, w2_ref[:], preferred_element_type=jnp.float32) + b2_ref[:],
        0.0)
    out_ref[:] = (jnp.dot(h, wout_ref[:], preferred_element_type=jnp.float32)
                  + bout_ref[:])


def _tc_mlp(u_emb, i_emb, W1, b1, W2, b2, Wout, bout, tile=4096):
    grid = (CB // tile,)
    row_spec = pl.BlockSpec((tile, EMB), lambda g: (g, 0))
    full = lambda shape: pl.BlockSpec(shape, lambda g: (0,) * len(shape))
    return pl.pallas_call(
        _mlp_body,
        grid=grid,
        in_specs=[
            row_spec, row_spec,
            pl.BlockSpec((EMB, 128), lambda g: (0, 0)),
            pl.BlockSpec((EMB, 128), lambda g: (1, 0)),
            full((1, 128)),
            full((128, 64)), full((1, 64)),
            full((64, 1)), full((1, 1)),
        ],
        out_specs=pl.BlockSpec((tile, 1), lambda g: (g, 0)),
        out_shape=jax.ShapeDtypeStruct((CB, 1), jnp.float32),
    )(u_emb, i_emb, W1, W1, b1.reshape(1, 128), W2, b2.reshape(1, 64),
      Wout, bout.reshape(1, 1))


@jax.jit
def kernel(users, items, user_table, movie_table, W1, b1, W2, b2, Wout, bout):
    embs = [_sc_gather(p, users, items, user_table, movie_table)
            for p in range(PIPE)]
    outs = [_tc_mlp(u, i, W1, b1, W2, b2, Wout, bout) for u, i in embs]
    return jnp.concatenate(outs, axis=0)
